# Initial kernel scaffold; baseline (speedup 1.0000x reference)
import jax
import jax.numpy as jnp
from jax.experimental import pallas as pl


def kernel(x, edge_index, W1_0, W1_1, b1, W2_0, W2_1, b2, Wl, bl):
    raise NotImplementedError("write your pallas kernel here")


# speedup vs baseline: 23.4758x; 23.4758x over previous
import functools

import jax
import jax.numpy as jnp
from jax import lax
from jax.experimental import pallas as pl
from jax.experimental.pallas import tpu as pltpu
from jax.experimental.pallas import tpu_sc as plsc

N = 10000
E = 320000
NP = 10240
F = 8
NC = 2
NS = 16
NW = NC * NS
EPT = E // NW
CH = 80
CHUNK = 128
DUMMY = N
RPS = NP // NS

_mesh = plsc.VectorSubcoreMesh(core_axis_name="c", subcore_axis_name="s")
_sc_params = pltpu.CompilerParams(use_tc_tiling_on_sc=False)


def _make_sc_prep():
    def body(erow_hbm, ecol_hbm, zeros_hbm, ones_hbm,
             rowp_hbm, colp_hbm, degp_hbm,
             row_v, col_v, rowp_v, colp_v, ones_v, deg_sh):
        cid = lax.axis_index("c")
        sid = lax.axis_index("s")
        wid = cid * NS + sid
        base = wid * EPT

        pltpu.sync_copy(zeros_hbm.at[pl.ds(sid * RPS, RPS)],
                        deg_sh.at[pl.ds(sid * RPS, RPS)])
        pltpu.sync_copy(ones_hbm, ones_v)
        plsc.subcore_barrier()

        pltpu.sync_copy(erow_hbm.at[pl.ds(base, EPT)],
                        row_v.at[pl.ds(0, EPT)])
        pltpu.sync_copy(ecol_hbm.at[pl.ds(base, EPT)],
                        col_v.at[pl.ds(0, EPT)])

        lane = lax.iota(jnp.int32, 16)

        def fill(ci, carry):
            for l in range(8):
                off = ci * CHUNK + l * 16
                r = row_v[pl.ds(off, 16)]
                c = col_v[pl.ds(off, 16)]
                valid = (off + lane) < EPT
                rp = jnp.where(valid & (r != c), r, DUMMY)
                cp = jnp.where(valid, c, DUMMY)
                rowp_v[ci, pl.ds(l * 16, 16)] = rp
                colp_v[ci, pl.ds(l * 16, 16)] = cp
            pltpu.sync_copy(ones_v, deg_sh.at[rowp_v.at[ci]], add=True)
            return carry

        lax.fori_loop(0, CH, fill, 0)

        pltpu.sync_copy(rowp_v, rowp_hbm.at[wid])
        pltpu.sync_copy(colp_v, colp_hbm.at[wid])
        plsc.subcore_barrier()
        pltpu.sync_copy(deg_sh.at[pl.ds(sid * RPS, RPS)],
                        degp_hbm.at[cid, pl.ds(sid * RPS, RPS)])

    return pl.kernel(
        body,
        out_type=[
            jax.ShapeDtypeStruct((NW, CH, CHUNK), jnp.int32),
            jax.ShapeDtypeStruct((NW, CH, CHUNK), jnp.int32),
            jax.ShapeDtypeStruct((NC, NP, F), jnp.float32),
        ],
        mesh=_mesh,
        scratch_types=[
            pltpu.VMEM((EPT + 240,), jnp.int32),
            pltpu.VMEM((EPT + 240,), jnp.int32),
            pltpu.VMEM((CH, CHUNK), jnp.int32),
            pltpu.VMEM((CH, CHUNK), jnp.int32),
            pltpu.VMEM((CHUNK, F), jnp.float32),
            pltpu.VMEM_SHARED((NP, F), jnp.float32),
        ],
        compiler_params=_sc_params,
    )


def _make_sc_prop():
    def body(rowp_hbm, colp_hbm, table_hbm, zeros_hbm, zp_hbm,
             rowp_v, colp_v, pay_v, z_sh, sem):
        cid = lax.axis_index("c")
        sid = lax.axis_index("s")
        wid = cid * NS + sid

        pltpu.sync_copy(zeros_hbm.at[pl.ds(sid * RPS, RPS)],
                        z_sh.at[pl.ds(sid * RPS, RPS)])
        pltpu.sync_copy(rowp_hbm.at[wid], rowp_v)
        pltpu.sync_copy(colp_hbm.at[wid], colp_v)
        plsc.subcore_barrier()

        def step(ci, carry):
            pltpu.async_copy(table_hbm.at[rowp_v.at[ci]], pay_v, sem).wait()
            pltpu.sync_copy(pay_v, z_sh.at[colp_v.at[ci]], add=True)
            return carry

        lax.fori_loop(0, CH, step, 0)

        plsc.subcore_barrier()
        pltpu.sync_copy(z_sh.at[pl.ds(sid * RPS, RPS)],
                        zp_hbm.at[cid, pl.ds(sid * RPS, RPS)])

    return pl.kernel(
        body,
        out_type=jax.ShapeDtypeStruct((NC, NP, F), jnp.float32),
        mesh=_mesh,
        scratch_types=[
            pltpu.VMEM((CH, CHUNK), jnp.int32),
            pltpu.VMEM((CH, CHUNK), jnp.int32),
            pltpu.VMEM((CHUNK, F), jnp.float32),
            pltpu.VMEM_SHARED((NP, F), jnp.float32),
            pltpu.SemaphoreType.DMA,
        ],
        compiler_params=_sc_params,
    )


_R = 1024


def _tca_body(x_ref, degp_ref, w_ref, b_ref, a1_ref, y1s_ref, dinv_ref):
    deg = degp_ref[0, :, 0] + degp_ref[1, :, 0]
    dinv = jnp.where(deg > 0.0, lax.rsqrt(deg), 0.0)
    o8 = jnp.dot(x_ref[...], w_ref[...],
                 preferred_element_type=jnp.float32) + b_ref[...]
    a1_ref[...] = o8[:, 0:F]
    y1s_ref[...] = o8[:, F:2 * F] * dinv[:, None]
    dinv_ref[...] = jnp.broadcast_to(dinv[:, None], (_R, F))


def _tcb_body(a1_ref, zp_ref, dinv_ref, h1_ref, h1s_ref):
    i = pl.program_id(0)
    dinv = dinv_ref[...]
    z1 = -dinv * (zp_ref[0] + zp_ref[1])
    h1 = jnp.maximum(a1_ref[...] + z1, 0.0)
    h1_ref[...] = h1
    rows = i * _R + lax.broadcasted_iota(jnp.int32, (_R, F), 0)
    h1s_ref[...] = jnp.where(rows < N, dinv * h1, 0.0)


def _tcc_body(h1_ref, zp_ref, dinv_ref, w2_ref, b2_ref, wl_ref, bl_ref,
              out_ref):
    z2 = -dinv_ref[...] * (zp_ref[0] + zp_ref[1])
    g8 = jnp.concatenate([h1_ref[...], z2], axis=1)
    g = jnp.maximum(
        jnp.dot(g8, w2_ref[...], preferred_element_type=jnp.float32)
        + b2_ref[...], 0.0)
    out_ref[...] = (jnp.dot(g, wl_ref[...],
                            preferred_element_type=jnp.float32)
                    + bl_ref[...])


def _row_spec(feat):
    return pl.BlockSpec((_R, feat), lambda i: (i, 0))


def _part_spec(feat):
    return pl.BlockSpec((NC, _R, feat), lambda i: (0, i, 0))


def _full_spec(a, b):
    return pl.BlockSpec((a, b), lambda i: (0, 0))


_GRID = NP // _R

_tca = pl.pallas_call(
    _tca_body,
    grid=(_GRID,),
    in_specs=[_row_spec(128), _part_spec(F), _full_spec(128, 2 * F),
              _full_spec(1, 2 * F)],
    out_specs=[_row_spec(F), _row_spec(F), _row_spec(F)],
    out_shape=[jax.ShapeDtypeStruct((NP, F), jnp.float32)] * 3,
)

_tcb = pl.pallas_call(
    _tcb_body,
    grid=(_GRID,),
    in_specs=[_row_spec(F), _part_spec(F), _row_spec(F)],
    out_specs=[_row_spec(F), _row_spec(F)],
    out_shape=[jax.ShapeDtypeStruct((NP, F), jnp.float32)] * 2,
)

_tcc = pl.pallas_call(
    _tcc_body,
    grid=(_GRID,),
    in_specs=[_row_spec(F), _part_spec(F), _row_spec(F),
              _full_spec(2 * F, 128), _full_spec(1, 128),
              _full_spec(128, 128), _full_spec(1, 128)],
    out_specs=_row_spec(128),
    out_shape=jax.ShapeDtypeStruct((NP, 128), jnp.float32),
)


@jax.jit
def kernel(x, edge_index, W1_0, W1_1, b1, W2_0, W2_1, b2, Wl, bl):
    f32 = jnp.float32
    x_pad = jnp.zeros((NP, 128), f32).at[:N].set(x)

    wcat = jnp.zeros((128, 2 * F), f32)
    wcat = wcat.at[:, 0:3].set(W1_0).at[:, F:F + 3].set(W1_1)
    bcat = jnp.zeros((1, 2 * F), f32).at[0, 0:3].set(b1)

    w2cat = jnp.zeros((2 * F, 128), f32)
    w2cat = w2cat.at[0:3].set(W2_0).at[F:F + 3].set(W2_1)
    b2r = jnp.reshape(b2, (1, 128))
    blr = jnp.reshape(bl, (1, 128))

    zeros_np = jnp.zeros((NP, F), f32)
    ones_ch = jnp.ones((CHUNK, F), f32)

    sc_prep = _make_sc_prep()
    sc_prop = _make_sc_prop()

    rowp, colp, degp = sc_prep(edge_index[0], edge_index[1],
                               zeros_np, ones_ch)
    a1, y1s, dinv = _tca(x_pad, degp, wcat, bcat)
    zp1 = sc_prop(rowp, colp, y1s, zeros_np)
    h1, h1s = _tcb(a1, zp1, dinv)
    zp2 = sc_prop(rowp, colp, h1s, zeros_np)
    out = _tcc(h1, zp2, dinv, w2cat, b2r, Wl, blr)
    return out[:N]


# speedup vs baseline: 31.4633x; 1.3402x over previous
import functools

import jax
import jax.numpy as jnp
from jax import lax
from jax.experimental import pallas as pl
from jax.experimental.pallas import tpu as pltpu
from jax.experimental.pallas import tpu_sc as plsc

N = 10000
E = 320000
NP = 10240
F = 8
NC = 2
NS = 16
NW = NC * NS
EPT = E // NW
CH = 80
CHUNK = 128
DUMMY = N
RPS = NP // NS

_mesh = plsc.VectorSubcoreMesh(core_axis_name="c", subcore_axis_name="s")
_sc_params = pltpu.CompilerParams(use_tc_tiling_on_sc=False)


def _make_sc_prep():
    def body(erow_hbm, ecol_hbm, zeros_hbm, ones_hbm,
             rowp_hbm, colp_hbm, degp_hbm,
             row_v, col_v, rowp_v, colp_v, ones_v, deg_sh):
        cid = lax.axis_index("c")
        sid = lax.axis_index("s")
        wid = cid * NS + sid
        base = wid * EPT

        pltpu.sync_copy(zeros_hbm.at[pl.ds(sid * RPS, RPS)],
                        deg_sh.at[pl.ds(sid * RPS, RPS)])
        pltpu.sync_copy(ones_hbm, ones_v)
        plsc.subcore_barrier()

        pltpu.sync_copy(erow_hbm.at[pl.ds(base, EPT)],
                        row_v.at[pl.ds(0, EPT)])
        pltpu.sync_copy(ecol_hbm.at[pl.ds(base, EPT)],
                        col_v.at[pl.ds(0, EPT)])

        lane = lax.iota(jnp.int32, 16)

        def fill(ci, carry):
            for l in range(8):
                off = ci * CHUNK + l * 16
                r = row_v[pl.ds(off, 16)]
                c = col_v[pl.ds(off, 16)]
                valid = (off + lane) < EPT
                rp = jnp.where(valid & (r != c), r, DUMMY)
                cp = jnp.where(valid, c, DUMMY)
                rowp_v[ci, pl.ds(l * 16, 16)] = rp
                colp_v[ci, pl.ds(l * 16, 16)] = cp
            pltpu.sync_copy(ones_v, deg_sh.at[rowp_v.at[ci]], add=True)
            return carry

        lax.fori_loop(0, CH, fill, 0)

        pltpu.sync_copy(rowp_v, rowp_hbm.at[wid])
        pltpu.sync_copy(colp_v, colp_hbm.at[wid])
        plsc.subcore_barrier()
        pltpu.sync_copy(deg_sh.at[pl.ds(sid * RPS, RPS)],
                        degp_hbm.at[cid, pl.ds(sid * RPS, RPS)])

    return pl.kernel(
        body,
        out_type=[
            jax.ShapeDtypeStruct((NW, CH, CHUNK), jnp.int32),
            jax.ShapeDtypeStruct((NW, CH, CHUNK), jnp.int32),
            jax.ShapeDtypeStruct((NC, NP, F), jnp.float32),
        ],
        mesh=_mesh,
        scratch_types=[
            pltpu.VMEM((EPT + 240,), jnp.int32),
            pltpu.VMEM((EPT + 240,), jnp.int32),
            pltpu.VMEM((CH, CHUNK), jnp.int32),
            pltpu.VMEM((CH, CHUNK), jnp.int32),
            pltpu.VMEM((CHUNK, F), jnp.float32),
            pltpu.VMEM_SHARED((NP, F), jnp.float32),
        ],
        compiler_params=_sc_params,
    )


_NBUF = 4


def _make_sc_prop():
    def body(rowp_hbm, colp_hbm, table_hbm, zeros_hbm, zp_hbm,
             rowp_v, colp_v, p0, p1, p2, p3, z_shared,
             g0, g1, g2, g3, s0, s1, s2, s3):
        pay = (p0, p1, p2, p3)
        gsem = (g0, g1, g2, g3)
        ssem = (s0, s1, s2, s3)
        cid = lax.axis_index("c")
        sid = lax.axis_index("s")
        wid = cid * NS + sid

        pltpu.sync_copy(zeros_hbm.at[pl.ds(sid * RPS, RPS)],
                        z_shared.at[pl.ds(sid * RPS, RPS)])
        pltpu.sync_copy(rowp_hbm.at[wid], rowp_v)
        pltpu.sync_copy(colp_hbm.at[wid], colp_v)
        plsc.subcore_barrier()

        for b in range(_NBUF):
            pltpu.async_copy(table_hbm.at[rowp_v.at[b]], pay[b], gsem[b])

        def step(t, carry):
            for b in range(_NBUF):
                ci = t * _NBUF + b
                pltpu.make_async_copy(table_hbm.at[rowp_v.at[ci]],
                                      pay[b], gsem[b]).wait()
                pltpu.async_copy(pay[b], z_shared.at[colp_v.at[ci]],
                                 ssem[b], add=True)
            for b in range(_NBUF):
                ci = t * _NBUF + _NBUF + b
                pltpu.make_async_copy(pay[b],
                                      z_shared.at[colp_v.at[ci]],
                                      ssem[b]).wait()
                pltpu.async_copy(table_hbm.at[rowp_v.at[ci]],
                                 pay[b], gsem[b])
            return carry

        lax.fori_loop(0, CH // _NBUF - 1, step, 0)

        for b in range(_NBUF):
            ci = CH - _NBUF + b
            pltpu.make_async_copy(table_hbm.at[rowp_v.at[ci]],
                                  pay[b], gsem[b]).wait()
            pltpu.async_copy(pay[b], z_shared.at[colp_v.at[ci]],
                             ssem[b], add=True)
        for b in range(_NBUF):
            pltpu.make_async_copy(pay[b], z_shared.at[colp_v.at[b]],
                                  ssem[b]).wait()

        plsc.subcore_barrier()
        pltpu.sync_copy(z_shared.at[pl.ds(sid * RPS, RPS)],
                        zp_hbm.at[cid, pl.ds(sid * RPS, RPS)])

    return pl.kernel(
        body,
        out_type=jax.ShapeDtypeStruct((NC, NP, F), jnp.float32),
        mesh=_mesh,
        scratch_types=[
            pltpu.VMEM((CH, CHUNK), jnp.int32),
            pltpu.VMEM((CH, CHUNK), jnp.int32),
            pltpu.VMEM((CHUNK, F), jnp.float32),
            pltpu.VMEM((CHUNK, F), jnp.float32),
            pltpu.VMEM((CHUNK, F), jnp.float32),
            pltpu.VMEM((CHUNK, F), jnp.float32),
            pltpu.VMEM_SHARED((NP, F), jnp.float32),
            pltpu.SemaphoreType.DMA,
            pltpu.SemaphoreType.DMA,
            pltpu.SemaphoreType.DMA,
            pltpu.SemaphoreType.DMA,
            pltpu.SemaphoreType.DMA,
            pltpu.SemaphoreType.DMA,
            pltpu.SemaphoreType.DMA,
            pltpu.SemaphoreType.DMA,
        ],
        compiler_params=_sc_params,
    )


_R = 1024


def _tca_body(x_ref, degp_ref, w_ref, b_ref, a1_ref, y1s_ref, dinv_ref):
    deg = degp_ref[0, :, 0] + degp_ref[1, :, 0]
    dinv = jnp.where(deg > 0.0, lax.rsqrt(deg), 0.0)
    o8 = jnp.dot(x_ref[...], w_ref[...],
                 preferred_element_type=jnp.float32) + b_ref[...]
    a1_ref[...] = o8[:, 0:F]
    y1s_ref[...] = o8[:, F:2 * F] * dinv[:, None]
    dinv_ref[...] = jnp.broadcast_to(dinv[:, None], (_R, F))


def _tcb_body(a1_ref, zp_ref, dinv_ref, h1_ref, h1s_ref):
    i = pl.program_id(0)
    dinv = dinv_ref[...]
    z1 = -dinv * (zp_ref[0] + zp_ref[1])
    h1 = jnp.maximum(a1_ref[...] + z1, 0.0)
    h1_ref[...] = h1
    rows = i * _R + lax.broadcasted_iota(jnp.int32, (_R, F), 0)
    h1s_ref[...] = jnp.where(rows < N, dinv * h1, 0.0)


def _tcc_body(h1_ref, zp_ref, dinv_ref, w2_ref, b2_ref, wl_ref, bl_ref,
              out_ref):
    z2 = -dinv_ref[...] * (zp_ref[0] + zp_ref[1])
    g8 = jnp.concatenate([h1_ref[...], z2], axis=1)
    g = jnp.maximum(
        jnp.dot(g8, w2_ref[...], preferred_element_type=jnp.float32)
        + b2_ref[...], 0.0)
    out_ref[...] = (jnp.dot(g, wl_ref[...],
                            preferred_element_type=jnp.float32)
                    + bl_ref[...])


def _row_spec(feat):
    return pl.BlockSpec((_R, feat), lambda i: (i, 0))


def _part_spec(feat):
    return pl.BlockSpec((NC, _R, feat), lambda i: (0, i, 0))


def _full_spec(a, b):
    return pl.BlockSpec((a, b), lambda i: (0, 0))


_GRID = NP // _R

_tca = pl.pallas_call(
    _tca_body,
    grid=(_GRID,),
    in_specs=[_row_spec(128), _part_spec(F), _full_spec(128, 2 * F),
              _full_spec(1, 2 * F)],
    out_specs=[_row_spec(F), _row_spec(F), _row_spec(F)],
    out_shape=[jax.ShapeDtypeStruct((NP, F), jnp.float32)] * 3,
)

_tcb = pl.pallas_call(
    _tcb_body,
    grid=(_GRID,),
    in_specs=[_row_spec(F), _part_spec(F), _row_spec(F)],
    out_specs=[_row_spec(F), _row_spec(F)],
    out_shape=[jax.ShapeDtypeStruct((NP, F), jnp.float32)] * 2,
)

_tcc = pl.pallas_call(
    _tcc_body,
    grid=(_GRID,),
    in_specs=[_row_spec(F), _part_spec(F), _row_spec(F),
              _full_spec(2 * F, 128), _full_spec(1, 128),
              _full_spec(128, 128), _full_spec(1, 128)],
    out_specs=_row_spec(128),
    out_shape=jax.ShapeDtypeStruct((NP, 128), jnp.float32),
)


@jax.jit
def kernel(x, edge_index, W1_0, W1_1, b1, W2_0, W2_1, b2, Wl, bl):
    f32 = jnp.float32
    x_pad = jnp.zeros((NP, 128), f32).at[:N].set(x)

    wcat = jnp.zeros((128, 2 * F), f32)
    wcat = wcat.at[:, 0:3].set(W1_0).at[:, F:F + 3].set(W1_1)
    bcat = jnp.zeros((1, 2 * F), f32).at[0, 0:3].set(b1)

    w2cat = jnp.zeros((2 * F, 128), f32)
    w2cat = w2cat.at[0:3].set(W2_0).at[F:F + 3].set(W2_1)
    b2r = jnp.reshape(b2, (1, 128))
    blr = jnp.reshape(bl, (1, 128))

    zeros_np = jnp.zeros((NP, F), f32)
    ones_ch = jnp.ones((CHUNK, F), f32)

    sc_prep = _make_sc_prep()
    sc_prop = _make_sc_prop()

    rowp, colp, degp = sc_prep(edge_index[0], edge_index[1],
                               zeros_np, ones_ch)
    a1, y1s, dinv = _tca(x_pad, degp, wcat, bcat)
    zp1 = sc_prop(rowp, colp, y1s, zeros_np)
    h1, h1s = _tcb(a1, zp1, dinv)
    zp2 = sc_prop(rowp, colp, h1s, zeros_np)
    out = _tcc(h1, zp2, dinv, w2cat, b2r, Wl, blr)
    return out[:N]


# speedup vs baseline: 33.9155x; 1.0779x over previous
import functools

import jax
import jax.numpy as jnp
from jax import lax
from jax.experimental import pallas as pl
from jax.experimental.pallas import tpu as pltpu
from jax.experimental.pallas import tpu_sc as plsc

N = 10000
E = 320000
NP = 10240
F = 8
NC = 2
NS = 16
NW = NC * NS
EPT = E // NW
CH = 80
CHUNK = 128
DUMMY = N
RPS = NP // NS

_mesh = plsc.VectorSubcoreMesh(core_axis_name="c", subcore_axis_name="s")
_sc_params = pltpu.CompilerParams(use_tc_tiling_on_sc=False)


def _make_sc_prep():
    def body(erow_hbm, ecol_hbm, zeros_hbm, ones_hbm,
             rowp_hbm, colp_hbm, degp_hbm,
             row_v, col_v, rowp_v, colp_v, ones_v, deg_sh, dsem):
        cid = lax.axis_index("c")
        sid = lax.axis_index("s")
        wid = cid * NS + sid
        base = wid * EPT

        pltpu.sync_copy(zeros_hbm.at[pl.ds(sid * RPS, RPS)],
                        deg_sh.at[pl.ds(sid * RPS, RPS)])
        pltpu.sync_copy(ones_hbm, ones_v)
        plsc.subcore_barrier()

        pltpu.sync_copy(erow_hbm.at[pl.ds(base, EPT)],
                        row_v.at[pl.ds(0, EPT)])
        pltpu.sync_copy(ecol_hbm.at[pl.ds(base, EPT)],
                        col_v.at[pl.ds(0, EPT)])

        lane = lax.iota(jnp.int32, 16)

        def fill(ci, carry):
            for l in range(8):
                off = ci * CHUNK + l * 16
                r = row_v[pl.ds(off, 16)]
                c = col_v[pl.ds(off, 16)]
                valid = (off + lane) < EPT
                rp = jnp.where(valid & (r != c), r, DUMMY)
                cp = jnp.where(valid, c, DUMMY)
                rowp_v[ci, pl.ds(l * 16, 16)] = rp
                colp_v[ci, pl.ds(l * 16, 16)] = cp
            pltpu.async_copy(ones_v, deg_sh.at[rowp_v.at[ci]], dsem,
                             add=True)
            return carry

        lax.fori_loop(0, CH, fill, 0)

        pltpu.sync_copy(rowp_v, rowp_hbm.at[wid])
        pltpu.sync_copy(colp_v, colp_hbm.at[wid])

        def drain(ci, carry):
            pltpu.make_async_copy(ones_v, deg_sh.at[rowp_v.at[ci]],
                                  dsem).wait()
            return carry

        lax.fori_loop(0, CH, drain, 0)
        plsc.subcore_barrier()
        pltpu.sync_copy(deg_sh.at[pl.ds(sid * RPS, RPS)],
                        degp_hbm.at[cid, pl.ds(sid * RPS, RPS)])

    return pl.kernel(
        body,
        out_type=[
            jax.ShapeDtypeStruct((NW, CH, CHUNK), jnp.int32),
            jax.ShapeDtypeStruct((NW, CH, CHUNK), jnp.int32),
            jax.ShapeDtypeStruct((NC, NP, F), jnp.float32),
        ],
        mesh=_mesh,
        scratch_types=[
            pltpu.VMEM((EPT + 240,), jnp.int32),
            pltpu.VMEM((EPT + 240,), jnp.int32),
            pltpu.VMEM((CH, CHUNK), jnp.int32),
            pltpu.VMEM((CH, CHUNK), jnp.int32),
            pltpu.VMEM((CHUNK, F), jnp.float32),
            pltpu.VMEM_SHARED((NP, F), jnp.float32),
            pltpu.SemaphoreType.DMA,
        ],
        compiler_params=_sc_params,
    )


_NBUF = 8


def _make_sc_prop():
    def body(rowp_hbm, colp_hbm, table_hbm, zeros_hbm, zp_hbm,
             rowp_v, colp_v, pay, z_shared, gsem, ssem):
        cid = lax.axis_index("c")
        sid = lax.axis_index("s")
        wid = cid * NS + sid

        pltpu.sync_copy(zeros_hbm.at[pl.ds(sid * RPS, RPS)],
                        z_shared.at[pl.ds(sid * RPS, RPS)])
        pltpu.sync_copy(rowp_hbm.at[wid], rowp_v)
        pltpu.sync_copy(colp_hbm.at[wid], colp_v)
        plsc.subcore_barrier()

        for b in range(_NBUF):
            pltpu.async_copy(table_hbm.at[rowp_v.at[b]], pay[b], gsem[b])

        def step(t, carry):
            for b in range(_NBUF):
                ci = t * _NBUF + b
                pltpu.make_async_copy(table_hbm.at[rowp_v.at[ci]],
                                      pay[b], gsem[b]).wait()
                pltpu.async_copy(pay[b], z_shared.at[colp_v.at[ci]],
                                 ssem[b], add=True)
            for b in range(_NBUF):
                ci = t * _NBUF + _NBUF + b
                pltpu.make_async_copy(pay[b],
                                      z_shared.at[colp_v.at[ci]],
                                      ssem[b]).wait()
                pltpu.async_copy(table_hbm.at[rowp_v.at[ci]],
                                 pay[b], gsem[b])
            return carry

        lax.fori_loop(0, CH // _NBUF - 1, step, 0)

        for b in range(_NBUF):
            ci = CH - _NBUF + b
            pltpu.make_async_copy(table_hbm.at[rowp_v.at[ci]],
                                  pay[b], gsem[b]).wait()
            pltpu.async_copy(pay[b], z_shared.at[colp_v.at[ci]],
                             ssem[b], add=True)
        for b in range(_NBUF):
            pltpu.make_async_copy(pay[b], z_shared.at[colp_v.at[b]],
                                  ssem[b]).wait()

        plsc.subcore_barrier()
        pltpu.sync_copy(z_shared.at[pl.ds(sid * RPS, RPS)],
                        zp_hbm.at[cid, pl.ds(sid * RPS, RPS)])

    return pl.kernel(
        body,
        out_type=jax.ShapeDtypeStruct((NC, NP, F), jnp.float32),
        mesh=_mesh,
        scratch_types=[
            pltpu.VMEM((CH, CHUNK), jnp.int32),
            pltpu.VMEM((CH, CHUNK), jnp.int32),
            [pltpu.VMEM((CHUNK, F), jnp.float32)] * _NBUF,
            pltpu.VMEM_SHARED((NP, F), jnp.float32),
            [pltpu.SemaphoreType.DMA] * _NBUF,
            [pltpu.SemaphoreType.DMA] * _NBUF,
        ],
        compiler_params=_sc_params,
    )


_R = 1024


def _tca_body(x_ref, degp_ref, w_ref, b_ref, a1_ref, y1s_ref, dinv_ref):
    deg = degp_ref[0, :, 0] + degp_ref[1, :, 0]
    dinv = jnp.where(deg > 0.0, lax.rsqrt(deg), 0.0)
    o8 = jnp.dot(x_ref[...], w_ref[...],
                 preferred_element_type=jnp.float32) + b_ref[...]
    a1_ref[...] = o8[:, 0:F]
    y1s_ref[...] = o8[:, F:2 * F] * dinv[:, None]
    dinv_ref[...] = jnp.broadcast_to(dinv[:, None], (_R, F))


def _tcb_body(a1_ref, zp_ref, dinv_ref, h1_ref, h1s_ref):
    i = pl.program_id(0)
    dinv = dinv_ref[...]
    z1 = -dinv * (zp_ref[0] + zp_ref[1])
    h1 = jnp.maximum(a1_ref[...] + z1, 0.0)
    h1_ref[...] = h1
    rows = i * _R + lax.broadcasted_iota(jnp.int32, (_R, F), 0)
    h1s_ref[...] = jnp.where(rows < N, dinv * h1, 0.0)


def _tcc_body(h1_ref, zp_ref, dinv_ref, w2_ref, b2_ref, wl_ref, bl_ref,
              out_ref):
    z2 = -dinv_ref[...] * (zp_ref[0] + zp_ref[1])
    g8 = jnp.concatenate([h1_ref[...], z2], axis=1)
    g = jnp.maximum(
        jnp.dot(g8, w2_ref[...], preferred_element_type=jnp.float32)
        + b2_ref[...], 0.0)
    out_ref[...] = (jnp.dot(g, wl_ref[...],
                            preferred_element_type=jnp.float32)
                    + bl_ref[...])


def _row_spec(feat):
    return pl.BlockSpec((_R, feat), lambda i: (i, 0))


def _part_spec(feat):
    return pl.BlockSpec((NC, _R, feat), lambda i: (0, i, 0))


def _full_spec(a, b):
    return pl.BlockSpec((a, b), lambda i: (0, 0))


_GRID = NP // _R

_tca = pl.pallas_call(
    _tca_body,
    grid=(_GRID,),
    in_specs=[_row_spec(128), _part_spec(F), _full_spec(128, 2 * F),
              _full_spec(1, 2 * F)],
    out_specs=[_row_spec(F), _row_spec(F), _row_spec(F)],
    out_shape=[jax.ShapeDtypeStruct((NP, F), jnp.float32)] * 3,
)

_tcb = pl.pallas_call(
    _tcb_body,
    grid=(_GRID,),
    in_specs=[_row_spec(F), _part_spec(F), _row_spec(F)],
    out_specs=[_row_spec(F), _row_spec(F)],
    out_shape=[jax.ShapeDtypeStruct((NP, F), jnp.float32)] * 2,
)

_tcc = pl.pallas_call(
    _tcc_body,
    grid=(_GRID,),
    in_specs=[_row_spec(F), _part_spec(F), _row_spec(F),
              _full_spec(2 * F, 128), _full_spec(1, 128),
              _full_spec(128, 128), _full_spec(1, 128)],
    out_specs=_row_spec(128),
    out_shape=jax.ShapeDtypeStruct((NP, 128), jnp.float32),
)


@jax.jit
def kernel(x, edge_index, W1_0, W1_1, b1, W2_0, W2_1, b2, Wl, bl):
    f32 = jnp.float32
    x_pad = jnp.zeros((NP, 128), f32).at[:N].set(x)

    wcat = jnp.zeros((128, 2 * F), f32)
    wcat = wcat.at[:, 0:3].set(W1_0).at[:, F:F + 3].set(W1_1)
    bcat = jnp.zeros((1, 2 * F), f32).at[0, 0:3].set(b1)

    w2cat = jnp.zeros((2 * F, 128), f32)
    w2cat = w2cat.at[0:3].set(W2_0).at[F:F + 3].set(W2_1)
    b2r = jnp.reshape(b2, (1, 128))
    blr = jnp.reshape(bl, (1, 128))

    zeros_np = jnp.zeros((NP, F), f32)
    ones_ch = jnp.ones((CHUNK, F), f32)

    sc_prep = _make_sc_prep()
    sc_prop = _make_sc_prop()

    rowp, colp, degp = sc_prep(edge_index[0], edge_index[1],
                               zeros_np, ones_ch)
    a1, y1s, dinv = _tca(x_pad, degp, wcat, bcat)
    zp1 = sc_prop(rowp, colp, y1s, zeros_np)
    h1, h1s = _tcb(a1, zp1, dinv)
    zp2 = sc_prop(rowp, colp, h1s, zeros_np)
    out = _tcc(h1, zp2, dinv, w2cat, b2r, Wl, blr)
    return out[:N]


# speedup vs baseline: 33.9686x; 1.0016x over previous
import functools

import jax
import jax.numpy as jnp
from jax import lax
from jax.experimental import pallas as pl
from jax.experimental.pallas import tpu as pltpu
from jax.experimental.pallas import tpu_sc as plsc

N = 10000
E = 320000
NP = 10240
F = 8
NC = 2
NS = 16
NW = NC * NS
EPT = E // NW
CH = 20
CHUNK = 512
DUMMY = N
RPS = NP // NS

_mesh = plsc.VectorSubcoreMesh(core_axis_name="c", subcore_axis_name="s")
_sc_params = pltpu.CompilerParams(use_tc_tiling_on_sc=False)


def _make_sc_prep():
    def body(erow_hbm, ecol_hbm, zeros_hbm, ones_hbm,
             rowp_hbm, colp_hbm, degp_hbm,
             row_v, col_v, rowp_v, colp_v, ones_v, deg_sh, dsem):
        cid = lax.axis_index("c")
        sid = lax.axis_index("s")
        wid = cid * NS + sid
        base = wid * EPT

        pltpu.sync_copy(zeros_hbm.at[pl.ds(sid * RPS, RPS)],
                        deg_sh.at[pl.ds(sid * RPS, RPS)])
        pltpu.sync_copy(ones_hbm, ones_v)
        plsc.subcore_barrier()

        pltpu.sync_copy(erow_hbm.at[pl.ds(base, EPT)],
                        row_v.at[pl.ds(0, EPT)])
        pltpu.sync_copy(ecol_hbm.at[pl.ds(base, EPT)],
                        col_v.at[pl.ds(0, EPT)])

        lane = lax.iota(jnp.int32, 16)

        def fill(ci, carry):
            for l in range(CHUNK // 16):
                off = ci * CHUNK + l * 16
                r = row_v[pl.ds(off, 16)]
                c = col_v[pl.ds(off, 16)]
                valid = (off + lane) < EPT
                rp = jnp.where(valid & (r != c), r, DUMMY)
                cp = jnp.where(valid, c, DUMMY)
                rowp_v[ci, pl.ds(l * 16, 16)] = rp
                colp_v[ci, pl.ds(l * 16, 16)] = cp
            pltpu.async_copy(ones_v, deg_sh.at[rowp_v.at[ci]], dsem,
                             add=True)
            return carry

        lax.fori_loop(0, CH, fill, 0)

        pltpu.sync_copy(rowp_v, rowp_hbm.at[wid])
        pltpu.sync_copy(colp_v, colp_hbm.at[wid])

        def drain(ci, carry):
            pltpu.make_async_copy(ones_v, deg_sh.at[rowp_v.at[ci]],
                                  dsem).wait()
            return carry

        lax.fori_loop(0, CH, drain, 0)
        plsc.subcore_barrier()
        pltpu.sync_copy(deg_sh.at[pl.ds(sid * RPS, RPS)],
                        degp_hbm.at[cid, pl.ds(sid * RPS, RPS)])

    return pl.kernel(
        body,
        out_type=[
            jax.ShapeDtypeStruct((NW, CH, CHUNK), jnp.int32),
            jax.ShapeDtypeStruct((NW, CH, CHUNK), jnp.int32),
            jax.ShapeDtypeStruct((NC, NP, F), jnp.float32),
        ],
        mesh=_mesh,
        scratch_types=[
            pltpu.VMEM((EPT + 240,), jnp.int32),
            pltpu.VMEM((EPT + 240,), jnp.int32),
            pltpu.VMEM((CH, CHUNK), jnp.int32),
            pltpu.VMEM((CH, CHUNK), jnp.int32),
            pltpu.VMEM((CHUNK, F), jnp.float32),
            pltpu.VMEM_SHARED((NP, F), jnp.float32),
            pltpu.SemaphoreType.DMA,
        ],
        compiler_params=_sc_params,
    )


_NBUF = 4


def _make_sc_prop():
    def body(rowp_hbm, colp_hbm, table_hbm, zeros_hbm, zp_hbm,
             rowp_v, colp_v, pay, z_shared, gsem, ssem):
        cid = lax.axis_index("c")
        sid = lax.axis_index("s")
        wid = cid * NS + sid

        pltpu.sync_copy(zeros_hbm.at[pl.ds(sid * RPS, RPS)],
                        z_shared.at[pl.ds(sid * RPS, RPS)])
        pltpu.sync_copy(rowp_hbm.at[wid], rowp_v)
        pltpu.sync_copy(colp_hbm.at[wid], colp_v)
        plsc.subcore_barrier()

        for b in range(_NBUF):
            pltpu.async_copy(table_hbm.at[rowp_v.at[b]], pay[b], gsem[b])

        def step(t, carry):
            for b in range(_NBUF):
                ci = t * _NBUF + b
                pltpu.make_async_copy(table_hbm.at[rowp_v.at[ci]],
                                      pay[b], gsem[b]).wait()
                pltpu.async_copy(pay[b], z_shared.at[colp_v.at[ci]],
                                 ssem[b], add=True)
            for b in range(_NBUF):
                ci = t * _NBUF + _NBUF + b
                pltpu.make_async_copy(pay[b],
                                      z_shared.at[colp_v.at[ci]],
                                      ssem[b]).wait()
                pltpu.async_copy(table_hbm.at[rowp_v.at[ci]],
                                 pay[b], gsem[b])
            return carry

        lax.fori_loop(0, CH // _NBUF - 1, step, 0)

        for b in range(_NBUF):
            ci = CH - _NBUF + b
            pltpu.make_async_copy(table_hbm.at[rowp_v.at[ci]],
                                  pay[b], gsem[b]).wait()
            pltpu.async_copy(pay[b], z_shared.at[colp_v.at[ci]],
                             ssem[b], add=True)
        for b in range(_NBUF):
            pltpu.make_async_copy(pay[b], z_shared.at[colp_v.at[b]],
                                  ssem[b]).wait()

        plsc.subcore_barrier()
        pltpu.sync_copy(z_shared.at[pl.ds(sid * RPS, RPS)],
                        zp_hbm.at[cid, pl.ds(sid * RPS, RPS)])

    return pl.kernel(
        body,
        out_type=jax.ShapeDtypeStruct((NC, NP, F), jnp.float32),
        mesh=_mesh,
        scratch_types=[
            pltpu.VMEM((CH, CHUNK), jnp.int32),
            pltpu.VMEM((CH, CHUNK), jnp.int32),
            [pltpu.VMEM((CHUNK, F), jnp.float32)] * _NBUF,
            pltpu.VMEM_SHARED((NP, F), jnp.float32),
            [pltpu.SemaphoreType.DMA] * _NBUF,
            [pltpu.SemaphoreType.DMA] * _NBUF,
        ],
        compiler_params=_sc_params,
    )


_R = 1024


def _tca_body(x_ref, degp_ref, w_ref, b_ref, a1_ref, y1s_ref, dinv_ref):
    deg = degp_ref[0, :, 0] + degp_ref[1, :, 0]
    dinv = jnp.where(deg > 0.0, lax.rsqrt(deg), 0.0)
    o8 = jnp.dot(x_ref[...], w_ref[...],
                 preferred_element_type=jnp.float32) + b_ref[...]
    a1_ref[...] = o8[:, 0:F]
    y1s_ref[...] = o8[:, F:2 * F] * dinv[:, None]
    dinv_ref[...] = jnp.broadcast_to(dinv[:, None], (_R, F))


def _tcb_body(a1_ref, zp_ref, dinv_ref, h1_ref, h1s_ref):
    i = pl.program_id(0)
    dinv = dinv_ref[...]
    z1 = -dinv * (zp_ref[0] + zp_ref[1])
    h1 = jnp.maximum(a1_ref[...] + z1, 0.0)
    h1_ref[...] = h1
    rows = i * _R + lax.broadcasted_iota(jnp.int32, (_R, F), 0)
    h1s_ref[...] = jnp.where(rows < N, dinv * h1, 0.0)


def _tcc_body(h1_ref, zp_ref, dinv_ref, w2_ref, b2_ref, wl_ref, bl_ref,
              out_ref):
    z2 = -dinv_ref[...] * (zp_ref[0] + zp_ref[1])
    g8 = jnp.concatenate([h1_ref[...], z2], axis=1)
    g = jnp.maximum(
        jnp.dot(g8, w2_ref[...], preferred_element_type=jnp.float32)
        + b2_ref[...], 0.0)
    out_ref[...] = (jnp.dot(g, wl_ref[...],
                            preferred_element_type=jnp.float32)
                    + bl_ref[...])


def _row_spec(feat):
    return pl.BlockSpec((_R, feat), lambda i: (i, 0))


def _part_spec(feat):
    return pl.BlockSpec((NC, _R, feat), lambda i: (0, i, 0))


def _full_spec(a, b):
    return pl.BlockSpec((a, b), lambda i: (0, 0))


_GRID = NP // _R

_tca = pl.pallas_call(
    _tca_body,
    grid=(_GRID,),
    in_specs=[_row_spec(128), _part_spec(F), _full_spec(128, 2 * F),
              _full_spec(1, 2 * F)],
    out_specs=[_row_spec(F), _row_spec(F), _row_spec(F)],
    out_shape=[jax.ShapeDtypeStruct((NP, F), jnp.float32)] * 3,
)

_tcb = pl.pallas_call(
    _tcb_body,
    grid=(_GRID,),
    in_specs=[_row_spec(F), _part_spec(F), _row_spec(F)],
    out_specs=[_row_spec(F), _row_spec(F)],
    out_shape=[jax.ShapeDtypeStruct((NP, F), jnp.float32)] * 2,
)

_tcc = pl.pallas_call(
    _tcc_body,
    grid=(_GRID,),
    in_specs=[_row_spec(F), _part_spec(F), _row_spec(F),
              _full_spec(2 * F, 128), _full_spec(1, 128),
              _full_spec(128, 128), _full_spec(1, 128)],
    out_specs=_row_spec(128),
    out_shape=jax.ShapeDtypeStruct((NP, 128), jnp.float32),
)


@jax.jit
def kernel(x, edge_index, W1_0, W1_1, b1, W2_0, W2_1, b2, Wl, bl):
    f32 = jnp.float32
    x_pad = jnp.zeros((NP, 128), f32).at[:N].set(x)

    wcat = jnp.zeros((128, 2 * F), f32)
    wcat = wcat.at[:, 0:3].set(W1_0).at[:, F:F + 3].set(W1_1)
    bcat = jnp.zeros((1, 2 * F), f32).at[0, 0:3].set(b1)

    w2cat = jnp.zeros((2 * F, 128), f32)
    w2cat = w2cat.at[0:3].set(W2_0).at[F:F + 3].set(W2_1)
    b2r = jnp.reshape(b2, (1, 128))
    blr = jnp.reshape(bl, (1, 128))

    zeros_np = jnp.zeros((NP, F), f32)
    ones_ch = jnp.ones((CHUNK, F), f32)

    sc_prep = _make_sc_prep()
    sc_prop = _make_sc_prop()

    rowp, colp, degp = sc_prep(edge_index[0], edge_index[1],
                               zeros_np, ones_ch)
    a1, y1s, dinv = _tca(x_pad, degp, wcat, bcat)
    zp1 = sc_prop(rowp, colp, y1s, zeros_np)
    h1, h1s = _tcb(a1, zp1, dinv)
    zp2 = sc_prop(rowp, colp, h1s, zeros_np)
    out = _tcc(h1, zp2, dinv, w2cat, b2r, Wl, blr)
    return out[:N]
